# Initial kernel scaffold; baseline (speedup 1.0000x reference)
import jax
import jax.numpy as jnp
from jax.experimental import pallas as pl


def kernel(users, pos_items, neg_items, epoch, user_table, item_table, weight1, weight2, weight3, beta_uD, beta_iD, ii_neighbor_mat, ii_constraint_mat):
    raise NotImplementedError("write your pallas kernel here")


# speedup vs baseline: 5.0163x; 5.0163x over previous
import jax
import jax.numpy as jnp
from jax import lax
from jax.experimental import pallas as pl
from jax.experimental.pallas import tpu as pltpu
from jax.experimental.pallas import tpu_sc as plsc

D = 64
B = 4096
NNEG = 50
KNBR = 10
W1 = 1e-6
W2 = 1.0
W3 = 1e-6
W4 = 1.0
NEG_WEIGHT = 10.0
GAMMA_REG = 1e-4
LAMBDA_ = 1e-3

NC = 2
NS = 16
L = 16
NW = NC * NS
BPW = B // NW
NV = BPW // L

_INTERPRET = False


def _sc_mesh():
    return plsc.VectorSubcoreMesh(
        core_axis_name="c", subcore_axis_name="s",
        num_cores=NC, num_subcores=NS)


def _sc_body(users_h, pos_h, negf_h, neg0_h, nbrpos_h, utab_h, itab_h,
             bu_h, bi_h, nbrf_h, simf_h,
             ps_o, n0_o, bu_o, bip_o, negs_o, bin_o, inner_o, sim_o,
             u_idx, p_idx, c_idx, i_idx, u_rows, a_rows, f_v, score_c,
             sem):
    wid = lax.axis_index("s") * NC + lax.axis_index("c")
    base = wid * BPW

    lane = lax.iota(jnp.int32, L)

    def pair_score(bl, j):
        acc = u_rows[bl, pl.ds(0, L)] * a_rows[j, pl.ds(0, L)]
        for k in range(1, D // L):
            acc = acc + u_rows[bl, pl.ds(k * L, L)] * a_rows[j, pl.ds(k * L, L)]
        for sh in (8, 4, 2, 1):
            acc = acc + acc[lane ^ sh]
        return acc

    def chunk_scores(flat0, q):
        def body(j0, carry):
            vec = jnp.zeros((L,), jnp.float32)
            for jj in range(L):
                j = j0 * L + jj
                bl = (flat0 + j) // q if q > 1 else j
                vec = jnp.where(lane == jj, pair_score(bl, j), vec)
            score_c[pl.ds(j0 * L, L)] = vec
            return carry
        lax.fori_loop(0, NV, body, 0)

    pltpu.sync_copy(users_h.at[pl.ds(base, BPW)], u_idx)
    pltpu.sync_copy(pos_h.at[pl.ds(base, BPW)], p_idx)
    pltpu.async_copy(utab_h.at[u_idx], u_rows, sem).wait()
    pltpu.async_copy(bu_h.at[u_idx], f_v, sem).wait()
    pltpu.sync_copy(f_v, bu_o.at[pl.ds(base, BPW)])
    pltpu.async_copy(bi_h.at[p_idx], f_v, sem).wait()
    pltpu.sync_copy(f_v, bip_o.at[pl.ds(base, BPW)])

    pltpu.async_copy(itab_h.at[p_idx], a_rows, sem).wait()
    chunk_scores(0, 1)
    pltpu.sync_copy(score_c, ps_o.at[pl.ds(base, BPW)])

    pltpu.sync_copy(neg0_h.at[pl.ds(base, BPW)], c_idx)
    pltpu.async_copy(itab_h.at[c_idx], a_rows, sem).wait()
    chunk_scores(0, 1)
    pltpu.sync_copy(score_c, n0_o.at[pl.ds(base, BPW)])

    fbase_n = base * NNEG

    def neg_body(c, carry):
        off = c * BPW
        pltpu.sync_copy(negf_h.at[pl.ds(fbase_n + off, BPW)], c_idx)
        pltpu.async_copy(itab_h.at[c_idx], a_rows, sem).wait()
        chunk_scores(off, NNEG)
        pltpu.sync_copy(score_c, negs_o.at[pl.ds(fbase_n + off, BPW)])
        pltpu.async_copy(bi_h.at[c_idx], f_v, sem).wait()
        pltpu.sync_copy(f_v, bin_o.at[pl.ds(fbase_n + off, BPW)])
        return carry

    lax.fori_loop(0, NNEG, neg_body, 0)

    fbase_k = base * KNBR

    def nbr_body(c, carry):
        off = c * BPW
        pltpu.sync_copy(nbrpos_h.at[pl.ds(fbase_k + off, BPW)], c_idx)
        pltpu.async_copy(nbrf_h.at[c_idx], i_idx, sem).wait()
        pltpu.async_copy(itab_h.at[i_idx], a_rows, sem).wait()
        chunk_scores(off, KNBR)
        pltpu.sync_copy(score_c, inner_o.at[pl.ds(fbase_k + off, BPW)])
        pltpu.async_copy(simf_h.at[c_idx], f_v, sem).wait()
        pltpu.sync_copy(f_v, sim_o.at[pl.ds(fbase_k + off, BPW)])
        return carry

    lax.fori_loop(0, KNBR, nbr_body, 0)


def _sc_call(users, pos, negf, neg0, nbrpos, utab, itab, bu, bi, nbrf, simf):
    f32 = jnp.float32
    out_type = [
        jax.ShapeDtypeStruct((B,), f32),
        jax.ShapeDtypeStruct((B,), f32),
        jax.ShapeDtypeStruct((B,), f32),
        jax.ShapeDtypeStruct((B,), f32),
        jax.ShapeDtypeStruct((B * NNEG,), f32),
        jax.ShapeDtypeStruct((B * NNEG,), f32),
        jax.ShapeDtypeStruct((B * KNBR,), f32),
        jax.ShapeDtypeStruct((B * KNBR,), f32),
    ]
    scratch_types = [
        pltpu.VMEM((BPW,), jnp.int32),
        pltpu.VMEM((BPW,), jnp.int32),
        pltpu.VMEM((BPW,), jnp.int32),
        pltpu.VMEM((BPW,), jnp.int32),
        pltpu.VMEM((BPW, D), f32),
        pltpu.VMEM((BPW, D), f32),
        pltpu.VMEM((BPW,), f32),
        pltpu.VMEM((BPW,), f32),
        pltpu.SemaphoreType.DMA,
    ]
    fn = pl.kernel(_sc_body, out_type=out_type, mesh=_sc_mesh(),
                   scratch_types=scratch_types,
                   compiler_params=pltpu.CompilerParams(
                       use_tc_tiling_on_sc=False),
                   interpret=_INTERPRET)
    return fn(users, pos, negf, neg0, nbrpos, utab, itab, bu, bi, nbrf, simf)


_RB = 2000
_GN = 100000 // _RB


def _norm_body(u_ref, i_ref, o_ref, acc_ref):
    step = pl.program_id(0)

    @pl.when(step == 0)
    def _():
        acc_ref[0] = 0.0

    u = u_ref[...]
    it = i_ref[...]
    acc_ref[0] += jnp.sum(u * u) + jnp.sum(it * it)

    @pl.when(step == _GN - 1)
    def _():
        o_ref[...] = jnp.full((1, 1), 0.5 * acc_ref[0], jnp.float32)


def _tc_norm(utab, itab):
    return pl.pallas_call(
        _norm_body,
        grid=(_GN,),
        in_specs=[pl.BlockSpec((_RB, D), lambda i: (i, 0)),
                  pl.BlockSpec((_RB, D), lambda i: (i, 0))],
        out_specs=pl.BlockSpec((1, 1), lambda i: (0, 0)),
        out_shape=jax.ShapeDtypeStruct((1, 1), jnp.float32),
        scratch_shapes=[pltpu.SMEM((1,), jnp.float32)],
        interpret=_INTERPRET,
    )(utab, itab)


def _softplus(x):
    return jnp.maximum(x, 0.0) + jnp.log1p(jnp.exp(-jnp.abs(x)))


def _loss_body(ps_ref, n0_ref, bu_ref, bip_ref, negs_ref, bin_ref, bu2_ref,
               inner_ref, sim_ref, w_ref, o_ref):
    ps = ps_ref[...]
    n0 = n0_ref[...]
    pw = W1 + W2 * bu_ref[...] * bip_ref[...]
    pos_sum = jnp.sum(pw * _softplus(-ps))
    nw = W3 + W4 * bu2_ref[...] * bin_ref[...]
    neg_sum = jnp.sum(nw * _softplus(negs_ref[...]))
    loss = pos_sum + (NEG_WEIGHT / NNEG) * neg_sum
    diff = ps - n0
    sp_beta = jnp.mean(jnp.exp(4.0 * diff))
    g_loss = jnp.sum(jnp.logaddexp(0.0, sp_beta * (-diff))) / sp_beta
    w = w_ref[0, 0]
    loss_l = w * loss + (1.0 - w) * g_loss
    loss_i = jnp.sum(sim_ref[...] * _softplus(-inner_ref[...]))
    o_ref[...] = jnp.full((1, 1), loss_l + LAMBDA_ * loss_i, jnp.float32)


def _tc_loss(ps, n0, bu, bip, negs, binv, inner, sim, w):
    args = (ps.reshape(B // 128, 128), n0.reshape(B // 128, 128),
            bu.reshape(B // 128, 128), bip.reshape(B // 128, 128),
            negs.reshape(B, NNEG), binv.reshape(B, NNEG),
            bu.reshape(B, 1),
            inner.reshape(B * KNBR // 128, 128),
            sim.reshape(B * KNBR // 128, 128), w.reshape(1, 1))
    return pl.pallas_call(
        _loss_body,
        out_shape=jax.ShapeDtypeStruct((1, 1), jnp.float32),
        interpret=_INTERPRET,
    )(*args)


def kernel(users, pos_items, neg_items, epoch, user_table, item_table,
           weight1, weight2, weight3, beta_uD, beta_iD,
           ii_neighbor_mat, ii_constraint_mat):
    users = users.astype(jnp.int32)
    pos = pos_items.astype(jnp.int32)
    negf = neg_items.reshape(-1).astype(jnp.int32)
    neg0 = neg_items[:, 0].astype(jnp.int32)
    nbrf = ii_neighbor_mat.reshape(-1).astype(jnp.int32)
    simf = ii_constraint_mat.reshape(-1)
    nbrpos = (pos[:, None] * KNBR
              + jnp.arange(KNBR, dtype=jnp.int32)[None, :]).reshape(-1)

    ps, n0, bu, bip, negs, binv, inner, sim = _sc_call(
        users, pos, negf, neg0, nbrpos, user_table, item_table,
        beta_uD, beta_iD, nbrf, simf)

    norm = _tc_norm(user_table, item_table)
    w = jnp.minimum(jnp.float32(1.0), jnp.float32(epoch) / 30.0)
    loss_l = _tc_loss(ps, n0, bu, bip, negs, binv, inner, sim, w)

    wsq = 0.5 * (weight1 * weight1 + weight2 * weight2 + weight3 * weight3)
    return loss_l[0, 0] + GAMMA_REG * (norm[0, 0] + wsq)


# speedup vs baseline: 9.1808x; 1.8302x over previous
import jax
import jax.numpy as jnp
from jax import lax
from jax.experimental import pallas as pl
from jax.experimental.pallas import tpu as pltpu
from jax.experimental.pallas import tpu_sc as plsc

D = 64
B = 4096
NNEG = 50
KNBR = 10
NITEM = 100000
W1 = 1e-6
W2 = 1.0
W3 = 1e-6
W4 = 1.0
NEG_WEIGHT = 10.0
GAMMA_REG = 1e-4
LAMBDA_ = 1e-3

NC = 2
NS = 16
L = 16
NW = NC * NS
BPW = B // NW
NV = BPW // L

_SC_PARAMS = pltpu.CompilerParams(use_tc_tiling_on_sc=False)


def _sc_mesh():
    return plsc.VectorSubcoreMesh(
        core_axis_name="c", subcore_axis_name="s",
        num_cores=NC, num_subcores=NS)


def _mk_pair_score(u_rows, rows_ref, lane):
    def pair_score(bl, j):
        acc = u_rows[bl, pl.ds(0, L)] * rows_ref[j, pl.ds(0, L)]
        for k in range(1, D // L):
            acc = acc + (u_rows[bl, pl.ds(k * L, L)]
                         * rows_ref[j, pl.ds(k * L, L)])
        for sh in (8, 4, 2, 1):
            acc = acc + acc[lane ^ sh]
        return acc
    return pair_score


def _chunk_scores(u_rows, rows_ref, score_ref, lane, flat0, q):
    pair_score = _mk_pair_score(u_rows, rows_ref, lane)

    def body(j0, carry):
        vec = jnp.zeros((L,), jnp.float32)
        for jj in range(L):
            j = j0 * L + jj
            bl = (flat0 + j) // q if q > 1 else j
            vec = jnp.where(lane == jj, pair_score(bl, j), vec)
        score_ref[pl.ds(j0 * L, L)] = vec
        return carry
    lax.fori_loop(0, NV, body, 0)


def _sc_main_body(users_h, pos_h, negf_h, neg0_h, utab_h, itab_h, bu_h, bi_h,
                  ps_o, n0_o, bu_o, bip_o, negs_o, bin_o,
                  u_idx, p_idx, n0_idx, c_idx0, c_idx1, u_rows, a_rows,
                  b_rows, r0_rows, r1_rows, bu_v, bip_v, f0_v, f1_v, score_c,
                  semU, semA, semB, semBU, semBIP, semR0, semR1, semF0, semF1):
    wid = lax.axis_index("s") * NC + lax.axis_index("c")
    base = wid * BPW
    lane = lax.iota(jnp.int32, L)

    pltpu.sync_copy(users_h.at[pl.ds(base, BPW)], u_idx)
    pltpu.sync_copy(pos_h.at[pl.ds(base, BPW)], p_idx)
    pltpu.sync_copy(neg0_h.at[pl.ds(base, BPW)], n0_idx)
    cpU = pltpu.async_copy(utab_h.at[u_idx], u_rows, semU)
    cpA = pltpu.async_copy(itab_h.at[p_idx], a_rows, semA)
    cpB = pltpu.async_copy(itab_h.at[n0_idx], b_rows, semB)
    cpBU = pltpu.async_copy(bu_h.at[u_idx], bu_v, semBU)
    cpBIP = pltpu.async_copy(bi_h.at[p_idx], bip_v, semBIP)

    fbase_n = base * NNEG
    pltpu.sync_copy(negf_h.at[pl.ds(fbase_n, BPW)], c_idx0)
    pltpu.async_copy(itab_h.at[c_idx0], r0_rows, semR0)
    pltpu.async_copy(bi_h.at[c_idx0], f0_v, semF0)

    cpU.wait()
    cpA.wait()
    _chunk_scores(u_rows, a_rows, score_c, lane, 0, 1)
    pltpu.sync_copy(score_c, ps_o.at[pl.ds(base, BPW)])
    cpB.wait()
    _chunk_scores(u_rows, b_rows, score_c, lane, 0, 1)
    pltpu.sync_copy(score_c, n0_o.at[pl.ds(base, BPW)])
    cpBU.wait()
    pltpu.sync_copy(bu_v, bu_o.at[pl.ds(base, BPW)])
    cpBIP.wait()
    pltpu.sync_copy(bip_v, bip_o.at[pl.ds(base, BPW)])

    def consume(c, idx, rows, fv, semr, semf):
        off = c * BPW
        pltpu.make_async_copy(itab_h.at[idx], rows, semr).wait()
        _chunk_scores(u_rows, rows, score_c, lane, off, NNEG)
        pltpu.sync_copy(score_c, negs_o.at[pl.ds(fbase_n + off, BPW)])
        pltpu.make_async_copy(bi_h.at[idx], fv, semf).wait()
        pltpu.sync_copy(fv, bin_o.at[pl.ds(fbase_n + off, BPW)])

    def outer(cc, carry):
        e = cc * 2
        pltpu.sync_copy(negf_h.at[pl.ds(fbase_n + (e + 1) * BPW, BPW)],
                        c_idx1)
        pltpu.async_copy(itab_h.at[c_idx1], r1_rows, semR1)
        pltpu.async_copy(bi_h.at[c_idx1], f1_v, semF1)
        consume(e, c_idx0, r0_rows, f0_v, semR0, semF0)

        @pl.when(cc < NNEG // 2 - 1)
        def _():
            pltpu.sync_copy(negf_h.at[pl.ds(fbase_n + (e + 2) * BPW, BPW)],
                            c_idx0)
            pltpu.async_copy(itab_h.at[c_idx0], r0_rows, semR0)
            pltpu.async_copy(bi_h.at[c_idx0], f0_v, semF0)
        consume(e + 1, c_idx1, r1_rows, f1_v, semR1, semF1)
        return carry

    lax.fori_loop(0, NNEG // 2, outer, 0)


def _sc_main_call(users, pos, negf, neg0, utab, itab, bu, bi):
    f32 = jnp.float32
    out_type = [
        jax.ShapeDtypeStruct((B,), f32),
        jax.ShapeDtypeStruct((B,), f32),
        jax.ShapeDtypeStruct((B,), f32),
        jax.ShapeDtypeStruct((B,), f32),
        jax.ShapeDtypeStruct((B * NNEG,), f32),
        jax.ShapeDtypeStruct((B * NNEG,), f32),
    ]
    i32 = jnp.int32
    scratch_types = [
        pltpu.VMEM((BPW,), i32),
        pltpu.VMEM((BPW,), i32),
        pltpu.VMEM((BPW,), i32),
        pltpu.VMEM((BPW,), i32),
        pltpu.VMEM((BPW,), i32),
        pltpu.VMEM((BPW, D), f32),
        pltpu.VMEM((BPW, D), f32),
        pltpu.VMEM((BPW, D), f32),
        pltpu.VMEM((BPW, D), f32),
        pltpu.VMEM((BPW, D), f32),
        pltpu.VMEM((BPW,), f32),
        pltpu.VMEM((BPW,), f32),
        pltpu.VMEM((BPW,), f32),
        pltpu.VMEM((BPW,), f32),
        pltpu.VMEM((BPW,), f32),
    ] + [pltpu.SemaphoreType.DMA] * 9
    fn = pl.kernel(_sc_main_body, out_type=out_type, mesh=_sc_mesh(),
                   scratch_types=scratch_types, compiler_params=_SC_PARAMS)
    return fn(users, pos, negf, neg0, utab, itab, bu, bi)


def _sc_nbr_body(users_h, nbrpos_h, utab_h, itab_h, nbrf_h, simf_h,
                 inner_o, sim_o,
                 u_idx, u_rows, p0_idx, p1_idx, i0_idx, i1_idx,
                 r0_rows, r1_rows, s0_v, s1_v, score_c,
                 semU, semI0, semI1, semR0, semR1, semS0, semS1):
    wid = lax.axis_index("s") * NC + lax.axis_index("c")
    base = wid * BPW
    fbase = base * KNBR
    lane = lax.iota(jnp.int32, L)

    pltpu.sync_copy(users_h.at[pl.ds(base, BPW)], u_idx)
    cpU = pltpu.async_copy(utab_h.at[u_idx], u_rows, semU)

    pltpu.sync_copy(nbrpos_h.at[pl.ds(fbase, BPW)], p0_idx)
    cpI0 = pltpu.async_copy(nbrf_h.at[p0_idx], i0_idx, semI0)
    cpS0 = pltpu.async_copy(simf_h.at[p0_idx], s0_v, semS0)
    cpI0.wait()
    pltpu.async_copy(itab_h.at[i0_idx], r0_rows, semR0)
    cpU.wait()

    bufs = ((p0_idx, i0_idx, r0_rows, s0_v, semI0, semR0, semS0),
            (p1_idx, i1_idx, r1_rows, s1_v, semI1, semR1, semS1))

    def step(c, cur, nxt):
        p_c, i_c, r_c, s_c, semI_c, semR_c, semS_c = cur
        p_n, i_n, r_n, s_n, semI_n, semR_n, semS_n = nxt
        if c < KNBR - 1:
            pltpu.sync_copy(nbrpos_h.at[pl.ds(fbase + (c + 1) * BPW, BPW)],
                            p_n)
            pltpu.async_copy(nbrf_h.at[p_n], i_n, semI_n)
            pltpu.async_copy(simf_h.at[p_n], s_n, semS_n)
        off = c * BPW
        pltpu.make_async_copy(itab_h.at[i_c], r_c, semR_c).wait()
        _chunk_scores(u_rows, r_c, score_c, lane, off, KNBR)
        pltpu.sync_copy(score_c, inner_o.at[pl.ds(fbase + off, BPW)])
        if c < KNBR - 1:
            pltpu.make_async_copy(nbrf_h.at[p_n], i_n, semI_n).wait()
            pltpu.async_copy(itab_h.at[i_n], r_n, semR_n)
        pltpu.make_async_copy(simf_h.at[p_c], s_c, semS_c).wait()
        pltpu.sync_copy(s_c, sim_o.at[pl.ds(fbase + off, BPW)])

    for c in range(KNBR):
        step(c, bufs[c % 2], bufs[(c + 1) % 2])


def _sc_nbr_call(users, nbrpos, utab, itab, nbrf, simf):
    f32 = jnp.float32
    i32 = jnp.int32
    out_type = [
        jax.ShapeDtypeStruct((B * KNBR,), f32),
        jax.ShapeDtypeStruct((B * KNBR,), f32),
    ]
    scratch_types = [
        pltpu.VMEM((BPW,), i32),
        pltpu.VMEM((BPW, D), f32),
        pltpu.VMEM((BPW,), i32),
        pltpu.VMEM((BPW,), i32),
        pltpu.VMEM((BPW,), i32),
        pltpu.VMEM((BPW,), i32),
        pltpu.VMEM((BPW, D), f32),
        pltpu.VMEM((BPW, D), f32),
        pltpu.VMEM((BPW,), f32),
        pltpu.VMEM((BPW,), f32),
        pltpu.VMEM((BPW,), f32),
    ] + [pltpu.SemaphoreType.DMA] * 7
    fn = pl.kernel(_sc_nbr_body, out_type=out_type, mesh=_sc_mesh(),
                   scratch_types=scratch_types, compiler_params=_SC_PARAMS)
    return fn(users, nbrpos, utab, itab, nbrf, simf)


_RB = 2000
_GN = 100000 // _RB


def _norm_body(u_ref, i_ref, o_ref, acc_ref):
    step = pl.program_id(0)

    @pl.when(step == 0)
    def _():
        acc_ref[0] = 0.0

    u = u_ref[...]
    it = i_ref[...]
    acc_ref[0] += jnp.sum(u * u) + jnp.sum(it * it)

    @pl.when(step == _GN - 1)
    def _():
        o_ref[...] = jnp.full((1, 1), 0.5 * acc_ref[0], jnp.float32)


def _tc_norm(utab, itab):
    return pl.pallas_call(
        _norm_body,
        grid=(_GN,),
        in_specs=[pl.BlockSpec((_RB, D), lambda i: (i, 0)),
                  pl.BlockSpec((_RB, D), lambda i: (i, 0))],
        out_specs=pl.BlockSpec((1, 1), lambda i: (0, 0)),
        out_shape=jax.ShapeDtypeStruct((1, 1), jnp.float32),
        scratch_shapes=[pltpu.SMEM((1,), jnp.float32)],
    )(utab, itab)


def _softplus(x):
    return jnp.maximum(x, 0.0) + jnp.log1p(jnp.exp(-jnp.abs(x)))


def _loss_body(ps_ref, n0_ref, bu_ref, bip_ref, negs_ref, bin_ref, bu2_ref,
               inner_ref, sim_ref, w_ref, o_ref):
    ps = ps_ref[...]
    n0 = n0_ref[...]
    pw = W1 + W2 * bu_ref[...] * bip_ref[...]
    pos_sum = jnp.sum(pw * _softplus(-ps))
    nw = W3 + W4 * bu2_ref[...] * bin_ref[...]
    neg_sum = jnp.sum(nw * _softplus(negs_ref[...]))
    loss = pos_sum + (NEG_WEIGHT / NNEG) * neg_sum
    diff = ps - n0
    sp_beta = jnp.mean(jnp.exp(4.0 * diff))
    g_loss = jnp.sum(jnp.logaddexp(0.0, sp_beta * (-diff))) / sp_beta
    w = w_ref[0, 0]
    loss_l = w * loss + (1.0 - w) * g_loss
    loss_i = jnp.sum(sim_ref[...] * _softplus(-inner_ref[...]))
    o_ref[...] = jnp.full((1, 1), loss_l + LAMBDA_ * loss_i, jnp.float32)


def _tc_loss(ps, n0, bu, bip, negs, binv, inner, sim, w):
    args = (ps.reshape(B // 128, 128), n0.reshape(B // 128, 128),
            bu.reshape(B // 128, 128), bip.reshape(B // 128, 128),
            negs.reshape(B, NNEG), binv.reshape(B, NNEG),
            bu.reshape(B, 1),
            inner.reshape(B * KNBR // 128, 128),
            sim.reshape(B * KNBR // 128, 128), w.reshape(1, 1))
    return pl.pallas_call(
        _loss_body,
        out_shape=jax.ShapeDtypeStruct((1, 1), jnp.float32),
    )(*args)


def kernel(users, pos_items, neg_items, epoch, user_table, item_table,
           weight1, weight2, weight3, beta_uD, beta_iD,
           ii_neighbor_mat, ii_constraint_mat):
    users = users.astype(jnp.int32)
    pos = pos_items.astype(jnp.int32)
    negf = neg_items.reshape(-1).astype(jnp.int32)
    neg0 = neg_items[:, 0].astype(jnp.int32)
    nbrf = ii_neighbor_mat.T.reshape(-1).astype(jnp.int32)
    simf = ii_constraint_mat.T.reshape(-1)
    nbrpos = (pos[:, None]
              + jnp.arange(KNBR, dtype=jnp.int32)[None, :] * NITEM
              ).reshape(-1)

    ps, n0, bu, bip, negs, binv = _sc_main_call(
        users, pos, negf, neg0, user_table, item_table, beta_uD, beta_iD)
    inner, sim = _sc_nbr_call(users, nbrpos, user_table, item_table,
                              nbrf, simf)

    norm = _tc_norm(user_table, item_table)
    w = jnp.minimum(jnp.float32(1.0), jnp.float32(epoch) / 30.0)
    loss_l = _tc_loss(ps, n0, bu, bip, negs, binv, inner, sim, w)

    wsq = 0.5 * (weight1 * weight1 + weight2 * weight2 + weight3 * weight3)
    return loss_l[0, 0] + GAMMA_REG * (norm[0, 0] + wsq)
